# baseline (device time: 68531 ns/iter reference)
import jax
import jax.numpy as jnp
from jax import lax
from jax.experimental import pallas as pl
from jax.experimental.pallas import tpu as pltpu

N_DEV = 4
SQ = 1024
SKV = 1024
H_PER = 8
DH = 128
D_MODEL = 1024
D_HID = H_PER * DH
SCALE = 0.08838834764831843
BLK = 64
NCH = 4
CH = SQ // NCH


def kernel(x, Wq, K_ext, V_ext, Wo):
    x2 = x.reshape(SQ, D_MODEL)
    k2 = K_ext.reshape(SKV, D_HID)
    v2 = V_ext.reshape(SKV, D_HID)

    def body(x_ref, wq_ref, k_ref, v_ref, wo_ref, out_ref,
             ctx_ref, kbuf, vbuf, sbuf, rbuf, wq_vmem, wo_vmem,
             copy_sems, send_sems, recv_sems):
        my = lax.axis_index("i")
        a_part = my + 1 - 2 * lax.rem(my, 2)
        b_part = 3 - my

        col0 = pl.multiple_of(my * D_HID, D_HID)
        wq_copy = pltpu.make_async_copy(
            wq_ref.at[:, pl.ds(col0, D_HID)], wq_vmem, copy_sems.at[0])
        wo_copy = pltpu.make_async_copy(
            wo_ref.at[pl.ds(col0, D_HID), :], wo_vmem, copy_sems.at[1])
        wq_copy.start()
        wo_copy.start()

        barrier_sem = pltpu.get_barrier_semaphore()
        for nbr in (a_part, b_part):
            pl.semaphore_signal(
                barrier_sem, inc=1,
                device_id=(nbr,), device_id_type=pl.DeviceIdType.MESH,
            )
        pl.semaphore_wait(barrier_sem, 2)

        xb = x_ref[...].astype(jnp.bfloat16)
        wq_copy.wait()
        wq = wq_vmem[...].astype(jnp.bfloat16)
        q = lax.dot_general(
            xb, wq, (((1,), (0,)), ((), ())),
            preferred_element_type=jnp.float32,
        )
        q = (q * SCALE).astype(jnp.bfloat16)
        kbuf[...] = k_ref[...].astype(jnp.bfloat16)
        vbuf[...] = v_ref[...].astype(jnp.bfloat16)
        wo_copy.wait()
        wo = wo_vmem[...].astype(jnp.bfloat16)
        ones = jnp.ones((SKV, 1), jnp.bfloat16)

        def exchange(phase, c, target):
            rdma = pltpu.make_async_remote_copy(
                src_ref=sbuf.at[phase, c],
                dst_ref=rbuf.at[phase, c],
                send_sem=send_sems.at[phase, c],
                recv_sem=recv_sems.at[phase, c],
                device_id=(target,),
                device_id_type=pl.DeviceIdType.MESH,
            )
            rdma.start()
            return rdma

        def partner(phase, c):
            return a_part if (c + phase) % 2 == 0 else b_part

        def compute_chunk(c):
            r0 = c * CH
            qb = r0 // BLK + lax.broadcasted_iota(jnp.int32, (CH, SKV), 0) // BLK
            kb = lax.broadcasted_iota(jnp.int32, (CH, SKV), 1) // BLK
            mask = (qb == kb) | (kb == 0) | (lax.rem(qb + kb, 3) == 0)
            mask_add = jnp.where(mask, 0.0, -1e9).astype(jnp.float32)
            for h in range(H_PER):
                qh = q[r0:r0 + CH, h * DH:(h + 1) * DH]
                kh = kbuf[:, h * DH:(h + 1) * DH]
                s = lax.dot_general(
                    qh, kh, (((1,), (1,)), ((), ())),
                    preferred_element_type=jnp.float32,
                )
                e = jnp.exp(s + mask_add).astype(jnp.bfloat16)
                r = 1.0 / lax.dot_general(
                    e, ones, (((1,), (0,)), ((), ())),
                    preferred_element_type=jnp.float32,
                )
                ch = lax.dot_general(
                    e, vbuf[:, h * DH:(h + 1) * DH],
                    (((1,), (0,)), ((), ())),
                    preferred_element_type=jnp.float32,
                )
                ctx_ref[:, h * DH:(h + 1) * DH] = (ch * r).astype(jnp.bfloat16)
            return lax.dot_general(
                ctx_ref[...], wo, (((1,), (0,)), ((), ())),
                preferred_element_type=jnp.float32,
            )

        p1 = [None] * NCH
        p2 = [None] * NCH

        def finish_phase1(c):
            p1[c].wait()
            sbuf[1, c] = sbuf[0, c] + rbuf[0, c]
            p2[c] = exchange(1, c, partner(1, c))

        def finish_phase2(c):
            p2[c].wait()
            out_ref[c * CH:(c + 1) * CH] = (
                sbuf[1, c].astype(jnp.float32) + rbuf[1, c].astype(jnp.float32)
            )

        import os as _os
        if _os.environ.get("COMPUTE_ONLY"):
            for c in range(NCH):
                out_ref[c * CH:(c + 1) * CH] = compute_chunk(c)
            return

        for c in range(NCH):
            sbuf[0, c] = compute_chunk(c).astype(jnp.bfloat16)
            p1[c] = exchange(0, c, partner(0, c))
            if c >= 1:
                finish_phase1(c - 1)
            if c >= 2:
                finish_phase2(c - 2)
        finish_phase1(NCH - 1)
        finish_phase2(NCH - 2)
        finish_phase2(NCH - 1)

    out = pl.pallas_call(
        body,
        out_shape=jax.ShapeDtypeStruct((SQ, D_MODEL), jnp.float32),
        in_specs=[
            pl.BlockSpec(memory_space=pltpu.VMEM),
            pl.BlockSpec(memory_space=pl.ANY),
            pl.BlockSpec(memory_space=pltpu.VMEM),
            pl.BlockSpec(memory_space=pltpu.VMEM),
            pl.BlockSpec(memory_space=pl.ANY),
        ],
        out_specs=pl.BlockSpec(memory_space=pltpu.VMEM),
        scratch_shapes=[
            pltpu.VMEM((CH, D_HID), jnp.bfloat16),
            pltpu.VMEM((SKV, D_HID), jnp.bfloat16),
            pltpu.VMEM((SKV, D_HID), jnp.bfloat16),
            pltpu.VMEM((2, NCH, CH, D_MODEL), jnp.bfloat16),
            pltpu.VMEM((2, NCH, CH, D_MODEL), jnp.bfloat16),
            pltpu.VMEM((D_MODEL, D_HID), jnp.float32),
            pltpu.VMEM((D_HID, D_MODEL), jnp.float32),
            pltpu.SemaphoreType.DMA((2,)),
            pltpu.SemaphoreType.DMA((2, NCH)),
            pltpu.SemaphoreType.DMA((2, NCH)),
        ],
        compiler_params=pltpu.CompilerParams(
            collective_id=0, vmem_limit_bytes=100 * 1024 * 1024,
        ),
    )(x2, Wq, k2, v2, Wo)
    return out.reshape(1, SQ, D_MODEL)


# device time: 61579 ns/iter; 1.1129x vs baseline; 1.1129x over previous
import jax
import jax.numpy as jnp
from jax import lax
from jax.experimental import pallas as pl
from jax.experimental.pallas import tpu as pltpu

N_DEV = 4
SQ = 1024
SKV = 1024
H_PER = 8
DH = 128
D_MODEL = 1024
D_HID = H_PER * DH
SCALE = 0.08838834764831843
BLK = 64
NBLK = SQ // BLK

GROUP_BLOCKS = [[qb for qb in range(NBLK) if qb % 3 == r] for r in range(3)]
PERM_BLOCKS = GROUP_BLOCKS[0] + GROUP_BLOCKS[1] + GROUP_BLOCKS[2]
NB = [len(b) for b in GROUP_BLOCKS]
SZ = [n * BLK for n in NB]
GS = [0, SZ[0], SZ[0] + SZ[1]]
NGR = 3


def kernel(x, Wq, K_ext, V_ext, Wo):
    x2 = x.reshape(SQ, D_MODEL)
    k2 = K_ext.reshape(SKV, D_HID)
    v2 = V_ext.reshape(SKV, D_HID)

    def body(x_ref, wq_ref, k_ref, v_ref, wo_ref, out_ref,
             ctx_ref, xbuf, kbuf, vbuf, sbuf, rbuf, wq_vmem, wo_vmem,
             copy_sems, send_sems, recv_sems):
        my = lax.axis_index("i")
        a_part = my + 1 - 2 * lax.rem(my, 2)
        b_part = 3 - my

        col0 = pl.multiple_of(my * D_HID, D_HID)
        wq_copy = pltpu.make_async_copy(
            wq_ref.at[:, pl.ds(col0, D_HID)], wq_vmem, copy_sems.at[0])
        wo_copy = pltpu.make_async_copy(
            wo_ref.at[pl.ds(col0, D_HID), :], wo_vmem, copy_sems.at[1])
        wq_copy.start()
        wo_copy.start()

        barrier_sem = pltpu.get_barrier_semaphore()
        for nbr in (a_part, b_part):
            pl.semaphore_signal(
                barrier_sem, inc=1,
                device_id=(nbr,), device_id_type=pl.DeviceIdType.MESH,
            )
        pl.semaphore_wait(barrier_sem, 2)

        for j, qb in enumerate(PERM_BLOCKS):
            xbuf[j * BLK:(j + 1) * BLK] = (
                x_ref[qb * BLK:(qb + 1) * BLK].astype(jnp.bfloat16))
            kbuf[j * BLK:(j + 1) * BLK] = (
                k_ref[qb * BLK:(qb + 1) * BLK].astype(jnp.bfloat16))
            vbuf[j * BLK:(j + 1) * BLK] = (
                v_ref[qb * BLK:(qb + 1) * BLK].astype(jnp.bfloat16))
        wq_copy.wait()
        wq = wq_vmem[...].astype(jnp.bfloat16)
        q = lax.dot_general(
            xbuf[...], wq, (((1,), (0,)), ((), ())),
            preferred_element_type=jnp.float32,
        )
        q = (q * SCALE).astype(jnp.bfloat16)
        wo_copy.wait()
        wo = wo_vmem[...].astype(jnp.bfloat16)

        def compute_group(r):
            s0, n, nb = GS[r], SZ[r], NB[r]
            g = (3 - r) % 3
            gs, gn = GS[g], SZ[g]
            for h in range(H_PER):
                hs, he = h * DH, (h + 1) * DH
                qh = q[s0:s0 + n, hs:he]
                s1 = lax.dot_general(
                    qh, kbuf[gs:gs + gn, hs:he], (((1,), (1,)), ((), ())),
                    preferred_element_type=jnp.float32,
                )
                e1 = jnp.exp(s1)
                rsum = jnp.sum(e1, axis=-1, keepdims=True)
                ctx = lax.dot_general(
                    e1.astype(jnp.bfloat16), vbuf[gs:gs + gn, hs:he],
                    (((1,), (0,)), ((), ())),
                    preferred_element_type=jnp.float32,
                )
                if r != 0:
                    s0p = lax.dot_general(
                        qh, kbuf[0:BLK, hs:he], (((1,), (1,)), ((), ())),
                        preferred_element_type=jnp.float32,
                    )
                    e0 = jnp.exp(s0p)
                    rsum += jnp.sum(e0, axis=-1, keepdims=True)
                    ctx += lax.dot_general(
                        e0.astype(jnp.bfloat16), vbuf[0:BLK, hs:he],
                        (((1,), (0,)), ((), ())),
                        preferred_element_type=jnp.float32,
                    )
                    q3 = qh.reshape(nb, BLK, DH)
                    k3 = kbuf[s0:s0 + n, hs:he].reshape(nb, BLK, DH)
                    sd = lax.dot_general(
                        q3, k3, (((2,), (2,)), ((0,), (0,))),
                        preferred_element_type=jnp.float32,
                    )
                    ed = jnp.exp(sd)
                    rsum += jnp.sum(ed, axis=-1).reshape(n, 1)
                    v3 = vbuf[s0:s0 + n, hs:he].reshape(nb, BLK, DH)
                    cd = lax.dot_general(
                        ed.astype(jnp.bfloat16), v3,
                        (((2,), (1,)), ((0,), (0,))),
                        preferred_element_type=jnp.float32,
                    )
                    ctx += cd.reshape(n, DH)
                ctx_ref[:n, hs:he] = (ctx * (1.0 / rsum)).astype(jnp.bfloat16)
            return lax.dot_general(
                ctx_ref[:n, :], wo, (((1,), (0,)), ((), ())),
                preferred_element_type=jnp.float32,
            )

        def exchange(phase, c, target):
            rows = pl.ds(GS[c], SZ[c])
            rdma = pltpu.make_async_remote_copy(
                src_ref=sbuf.at[phase, rows],
                dst_ref=rbuf.at[phase, rows],
                send_sem=send_sems.at[phase, c],
                recv_sem=recv_sems.at[phase, c],
                device_id=(target,),
                device_id_type=pl.DeviceIdType.MESH,
            )
            rdma.start()
            return rdma

        def partner(phase, c):
            return a_part if (c + phase) % 2 == 0 else b_part

        p1 = [None] * NGR
        p2 = [None] * NGR

        def finish_phase1(c):
            s0, n = GS[c], SZ[c]
            p1[c].wait()
            sbuf[1, s0:s0 + n] = sbuf[0, s0:s0 + n] + rbuf[0, s0:s0 + n]
            p2[c] = exchange(1, c, partner(1, c))

        def finish_phase2(c):
            s0, n = GS[c], SZ[c]
            p2[c].wait()
            final = (sbuf[1, s0:s0 + n].astype(jnp.float32)
                     + rbuf[1, s0:s0 + n].astype(jnp.float32))
            for j, qb in enumerate(GROUP_BLOCKS[c]):
                out_ref[qb * BLK:(qb + 1) * BLK] = (
                    final[j * BLK:(j + 1) * BLK])

        import os as _os
        if _os.environ.get("COMPUTE_ONLY"):
            for c in range(NGR):
                final = compute_group(c)
                for j, qb in enumerate(GROUP_BLOCKS[c]):
                    out_ref[qb * BLK:(qb + 1) * BLK] = (
                        final[j * BLK:(j + 1) * BLK])
            return

        for c in range(NGR):
            s0, n = GS[c], SZ[c]
            sbuf[0, s0:s0 + n] = compute_group(c).astype(jnp.bfloat16)
            p1[c] = exchange(0, c, partner(0, c))
            if c >= 1:
                finish_phase1(c - 1)
            if c >= 2:
                finish_phase2(c - 2)
        finish_phase1(NGR - 1)
        finish_phase2(NGR - 2)
        finish_phase2(NGR - 1)

    out = pl.pallas_call(
        body,
        out_shape=jax.ShapeDtypeStruct((SQ, D_MODEL), jnp.float32),
        in_specs=[
            pl.BlockSpec(memory_space=pltpu.VMEM),
            pl.BlockSpec(memory_space=pl.ANY),
            pl.BlockSpec(memory_space=pltpu.VMEM),
            pl.BlockSpec(memory_space=pltpu.VMEM),
            pl.BlockSpec(memory_space=pl.ANY),
        ],
        out_specs=pl.BlockSpec(memory_space=pltpu.VMEM),
        scratch_shapes=[
            pltpu.VMEM((SZ[0], D_HID), jnp.bfloat16),
            pltpu.VMEM((SQ, D_MODEL), jnp.bfloat16),
            pltpu.VMEM((SKV, D_HID), jnp.bfloat16),
            pltpu.VMEM((SKV, D_HID), jnp.bfloat16),
            pltpu.VMEM((2, SQ, D_MODEL), jnp.bfloat16),
            pltpu.VMEM((2, SQ, D_MODEL), jnp.bfloat16),
            pltpu.VMEM((D_MODEL, D_HID), jnp.float32),
            pltpu.VMEM((D_HID, D_MODEL), jnp.float32),
            pltpu.SemaphoreType.DMA((2,)),
            pltpu.SemaphoreType.DMA((2, NGR)),
            pltpu.SemaphoreType.DMA((2, NGR)),
        ],
        compiler_params=pltpu.CompilerParams(
            collective_id=0, vmem_limit_bytes=100 * 1024 * 1024,
        ),
    )(x2, Wq, k2, v2, Wo)
    return out.reshape(1, SQ, D_MODEL)


# device time: 59900 ns/iter; 1.1441x vs baseline; 1.0280x over previous
import jax
import jax.numpy as jnp
from jax import lax
from jax.experimental import pallas as pl
from jax.experimental.pallas import tpu as pltpu

N_DEV = 4
SQ = 1024
SKV = 1024
H_PER = 8
DH = 128
D_MODEL = 1024
D_HID = H_PER * DH
SCALE = 0.08838834764831843
BLK = 64
NBLK = SQ // BLK

GROUP_BLOCKS = [[qb for qb in range(NBLK) if qb % 3 == r] for r in range(3)]
PERM_BLOCKS = GROUP_BLOCKS[0] + GROUP_BLOCKS[1] + GROUP_BLOCKS[2]
NB = [len(b) for b in GROUP_BLOCKS]
SZ = [n * BLK for n in NB]
GS = [0, SZ[0], SZ[0] + SZ[1]]
NGR = 3


def kernel(x, Wq, K_ext, V_ext, Wo):
    x2 = x.reshape(SQ, D_MODEL)
    k2 = K_ext.reshape(SKV, D_HID)
    v2 = V_ext.reshape(SKV, D_HID)

    def body(x_ref, wq_ref, k_ref, v_ref, wo_ref, out_ref,
             ctx_ref, xbuf, kbuf, vbuf, pbuf, rbuf0, hbuf, rbuf1, qbuf,
             wq_vmem, wo_vmem, copy_sems, send_sems, recv_sems):
        my = lax.axis_index("i")
        a_part = my + 1 - 2 * lax.rem(my, 2)
        b_part = 3 - my

        col0 = pl.multiple_of(my * D_HID, D_HID)
        wq_copy = pltpu.make_async_copy(
            wq_ref.at[:, pl.ds(col0, D_HID)], wq_vmem, copy_sems.at[0])
        wo_copy = pltpu.make_async_copy(
            wo_ref.at[pl.ds(col0, D_HID), :], wo_vmem, copy_sems.at[1])
        wq_copy.start()
        wo_copy.start()

        barrier_sem = pltpu.get_barrier_semaphore()
        for nbr in (a_part, b_part):
            pl.semaphore_signal(
                barrier_sem, inc=1,
                device_id=(nbr,), device_id_type=pl.DeviceIdType.MESH,
            )
        pl.semaphore_wait(barrier_sem, 2)

        for j, qb in enumerate(PERM_BLOCKS):
            xbuf[j * BLK:(j + 1) * BLK] = (
                x_ref[qb * BLK:(qb + 1) * BLK].astype(jnp.bfloat16))
            kbuf[j * BLK:(j + 1) * BLK] = (
                k_ref[qb * BLK:(qb + 1) * BLK].astype(jnp.bfloat16))
            vbuf[j * BLK:(j + 1) * BLK] = (
                v_ref[qb * BLK:(qb + 1) * BLK].astype(jnp.bfloat16))
        wq_copy.wait()
        wq = wq_vmem[...].astype(jnp.bfloat16)
        q = lax.dot_general(
            xbuf[...], wq, (((1,), (0,)), ((), ())),
            preferred_element_type=jnp.float32,
        )
        q = (q * SCALE).astype(jnp.bfloat16)
        wo_copy.wait()
        wo = wo_vmem[...].astype(jnp.bfloat16)

        def compute_group(r):
            s0, n, nb = GS[r], SZ[r], NB[r]
            g = (3 - r) % 3
            gs, gn = GS[g], SZ[g]
            for h in range(H_PER):
                hs, he = h * DH, (h + 1) * DH
                qh = q[s0:s0 + n, hs:he]
                s1 = lax.dot_general(
                    qh, kbuf[gs:gs + gn, hs:he], (((1,), (1,)), ((), ())),
                    preferred_element_type=jnp.float32,
                )
                e1 = jnp.exp(s1)
                rsum = jnp.sum(e1, axis=-1, keepdims=True)
                ctx = lax.dot_general(
                    e1.astype(jnp.bfloat16), vbuf[gs:gs + gn, hs:he],
                    (((1,), (0,)), ((), ())),
                    preferred_element_type=jnp.float32,
                )
                if r != 0:
                    s0p = lax.dot_general(
                        qh, kbuf[0:BLK, hs:he], (((1,), (1,)), ((), ())),
                        preferred_element_type=jnp.float32,
                    )
                    e0 = jnp.exp(s0p)
                    rsum += jnp.sum(e0, axis=-1, keepdims=True)
                    ctx += lax.dot_general(
                        e0.astype(jnp.bfloat16), vbuf[0:BLK, hs:he],
                        (((1,), (0,)), ((), ())),
                        preferred_element_type=jnp.float32,
                    )
                    q3 = qh.reshape(nb, BLK, DH)
                    k3 = kbuf[s0:s0 + n, hs:he].reshape(nb, BLK, DH)
                    sd = lax.dot_general(
                        q3, k3, (((2,), (2,)), ((0,), (0,))),
                        preferred_element_type=jnp.float32,
                    )
                    ed = jnp.exp(sd)
                    rsum += jnp.sum(ed, axis=-1).reshape(n, 1)
                    v3 = vbuf[s0:s0 + n, hs:he].reshape(nb, BLK, DH)
                    cd = lax.dot_general(
                        ed.astype(jnp.bfloat16), v3,
                        (((2,), (1,)), ((0,), (0,))),
                        preferred_element_type=jnp.float32,
                    )
                    ctx += cd.reshape(n, DH)
                ctx_ref[:n, hs:he] = (ctx * (1.0 / rsum)).astype(jnp.bfloat16)
            return lax.dot_general(
                ctx_ref[:n, :], wo, (((1,), (0,)), ((), ())),
                preferred_element_type=jnp.float32,
            )

        u = lax.rem(my, 2)
        v = lax.div(my, 2)
        hb = lax.rem(u + v, 2)

        def exchange(phase, c, src_ref, dst_ref, start, size, target):
            rows = pl.ds(pl.multiple_of(start, 16), size)
            rdma = pltpu.make_async_remote_copy(
                src_ref=src_ref.at[rows],
                dst_ref=dst_ref.at[rows],
                send_sem=send_sems.at[phase, c],
                recv_sem=recv_sems.at[phase, c],
                device_id=(target,),
                device_id_type=pl.DeviceIdType.MESH,
            )
            rdma.start()
            return rdma

        ph = [[None] * 4 for _ in range(NGR)]

        def rows_of(c):
            s0, n = GS[c], SZ[c]
            n2, n4 = n // 2, n // 4
            my_half = s0 + hb * n2
            oth_half = s0 + (1 - hb) * n2
            my_q = my_half + v * n4
            oth_q = my_half + (1 - v) * n4
            return s0, n, n2, n4, my_half, oth_half, my_q, oth_q

        def start_ph0(c):
            _, _, n2, _, _, oth_half, _, _ = rows_of(c)
            ph[c][0] = exchange(0, c, pbuf, rbuf0, oth_half, n2, a_part)

        def advance1(c):
            _, _, n2, n4, my_half, _, _, oth_q = rows_of(c)
            ph[c][0].wait()
            mh = pl.ds(pl.multiple_of(my_half, 16), n2)
            hbuf[mh] = pbuf[mh] + rbuf0[mh]
            ph[c][1] = exchange(1, c, hbuf, rbuf1, oth_q, n4, b_part)

        def advance2(c):
            _, _, n2, n4, my_half, _, my_q, _ = rows_of(c)
            ph[c][1].wait()
            mq = pl.ds(pl.multiple_of(my_q, 16), n4)
            qbuf[mq] = hbuf[mq] + rbuf1[mq]
            ph[c][2] = exchange(2, c, qbuf, qbuf, my_q, n4, b_part)
            ph[c][2].wait()
            ph[c][3] = exchange(3, c, qbuf, qbuf, my_half, n2, a_part)

        def finish(c):
            s0, n = GS[c], SZ[c]
            ph[c][3].wait()
            final = qbuf[s0:s0 + n].astype(jnp.float32)
            for j, qb in enumerate(GROUP_BLOCKS[c]):
                out_ref[qb * BLK:(qb + 1) * BLK] = (
                    final[j * BLK:(j + 1) * BLK])

        import os as _os
        if _os.environ.get("STOP_AFTER"):
            stop = int(_os.environ["STOP_AFTER"])
            for c in range(NGR):
                s0, n = GS[c], SZ[c]
                pbuf[s0:s0 + n] = q[s0:s0 + n, :] * 0.25
                start_ph0(c)
            if stop == 0:
                for c in range(NGR):
                    ph[c][0].wait()
            else:
                for c in range(NGR):
                    advance1(c)
                if stop == 1:
                    for c in range(NGR):
                        ph[c][1].wait()
                else:
                    for c in range(NGR):
                        advance2(c)
                    for c in range(NGR):
                        finish(c)
                    return
            out_ref[...] = rbuf0[...].astype(jnp.float32)
            return
        if _os.environ.get("COMM_ONLY"):
            def compute_group(r):
                return q[:SZ[r], :].astype(jnp.float32)
        if _os.environ.get("COMPUTE_ONLY"):
            for c in range(NGR):
                final = compute_group(c)
                for j, qb in enumerate(GROUP_BLOCKS[c]):
                    out_ref[qb * BLK:(qb + 1) * BLK] = (
                        final[j * BLK:(j + 1) * BLK])
            return

        for c in range(NGR):
            s0, n = GS[c], SZ[c]
            pbuf[s0:s0 + n] = compute_group(c).astype(jnp.bfloat16)
            start_ph0(c)
            if c >= 1:
                advance1(c - 1)
            if c >= 2:
                advance2(c - 2)
        advance1(NGR - 1)
        advance2(NGR - 2)
        finish(NGR - 3)
        advance2(NGR - 1)
        finish(NGR - 2)
        finish(NGR - 1)

    out = pl.pallas_call(
        body,
        out_shape=jax.ShapeDtypeStruct((SQ, D_MODEL), jnp.float32),
        in_specs=[
            pl.BlockSpec(memory_space=pltpu.VMEM),
            pl.BlockSpec(memory_space=pl.ANY),
            pl.BlockSpec(memory_space=pltpu.VMEM),
            pl.BlockSpec(memory_space=pltpu.VMEM),
            pl.BlockSpec(memory_space=pl.ANY),
        ],
        out_specs=pl.BlockSpec(memory_space=pltpu.VMEM),
        scratch_shapes=[
            pltpu.VMEM((SZ[0], D_HID), jnp.bfloat16),
            pltpu.VMEM((SQ, D_MODEL), jnp.bfloat16),
            pltpu.VMEM((SKV, D_HID), jnp.bfloat16),
            pltpu.VMEM((SKV, D_HID), jnp.bfloat16),
            pltpu.VMEM((SQ, D_MODEL), jnp.bfloat16),
            pltpu.VMEM((SQ, D_MODEL), jnp.bfloat16),
            pltpu.VMEM((SQ, D_MODEL), jnp.bfloat16),
            pltpu.VMEM((SQ, D_MODEL), jnp.bfloat16),
            pltpu.VMEM((SQ, D_MODEL), jnp.bfloat16),
            pltpu.VMEM((D_MODEL, D_HID), jnp.float32),
            pltpu.VMEM((D_HID, D_MODEL), jnp.float32),
            pltpu.SemaphoreType.DMA((2,)),
            pltpu.SemaphoreType.DMA((4, NGR)),
            pltpu.SemaphoreType.DMA((4, NGR)),
        ],
        compiler_params=pltpu.CompilerParams(
            collective_id=0, vmem_limit_bytes=100 * 1024 * 1024,
        ),
    )(x2, Wq, k2, v2, Wo)
    return out.reshape(1, SQ, D_MODEL)
